# baseline (speedup 1.0000x reference)
import jax
import jax.numpy as jnp
from jax import lax
from jax.experimental import pallas as pl
from jax.experimental.pallas import tpu as pltpu
from jax.experimental.pallas import tpu_sc as plsc

_VOCAB = 1000000
_D = 64
_B = 4096
_S = 200

_TBLK = 6144
_NBLK = 82
_LA = _TBLK * _NBLK
_LB = _TBLK * (_NBLK - 1)
_VPAD = 2 * _LA

_NC = 2
_NS = 16
_NW = _NC * _NS
_BPW = _B // _NW
_W = 16


def _tr_body(ta_ref, tb_ref, out_ref):
    out_ref[...] = jnp.transpose(
        jnp.concatenate([ta_ref[...], tb_ref[...]], axis=0))


@jax.jit
def _relayout_table(table_t):
    return pl.pallas_call(
        _tr_body,
        grid=(_NBLK,),
        in_specs=[
            pl.BlockSpec((_D, _TBLK), lambda g: (0, g)),
            pl.BlockSpec((_D, _TBLK), lambda g: (0, g + _NBLK - 1)),
        ],
        out_specs=pl.BlockSpec((_TBLK, 2 * _D), lambda g: (g, 0)),
        out_shape=jax.ShapeDtypeStruct((_LA, 2 * _D), jnp.float32),
    )(table_t, table_t)


def _pool_body(x_hbm, table_hbm, out_hbm, x_v, xt_v, acc_v, sems):
    wid = lax.axis_index("s") * _NC + lax.axis_index("c")
    pltpu.sync_copy(x_hbm.at[pl.ds(wid * _BPW, _BPW)], x_v)

    zeros = jnp.zeros((16,), jnp.float32)

    def zero_row(b, c):
        for ch in range(_D // 16):
            acc_v[b, pl.ds(ch * 16, 16)] = zeros
        return c

    lax.fori_loop(0, _BPW, zero_row, 0)

    iota = lax.iota(jnp.int32, 16)

    def fire(s, slot):
        pltpu.async_copy(table_hbm.at[xt_v.at[s]], acc_v, sems.at[slot],
                         add=True)

    def ring_wait(slot):
        pltpu.make_async_copy(table_hbm.at[xt_v.at[0]], acc_v,
                              sems.at[slot]).wait()

    def step(s, c):
        col_idx = jnp.full((16,), 0, jnp.int32) + s
        for ch in range(_BPW // 16):
            row_idx = iota + (ch * 16)
            v = plsc.load_gather(x_v, [row_idx, col_idx])
            t = v + v
            vp = jnp.where(v < _LA, t, t - (2 * _LB - 1))
            xt_v[s, pl.ds(ch * 16, 16)] = vp
        slot = lax.rem(s, _W)

        @pl.when(s >= _W)
        def _():
            ring_wait(slot)

        fire(s, slot)
        return c

    lax.fori_loop(0, _S, step, 0)
    for j in range(_W):
        ring_wait(j)

    pltpu.sync_copy(acc_v, out_hbm.at[pl.ds(wid * _BPW, _BPW)])


@jax.jit
def _pool(x, table_lin):
    mesh = plsc.VectorSubcoreMesh(
        core_axis_name="c", subcore_axis_name="s", num_cores=_NC,
        num_subcores=_NS)
    return pl.kernel(
        _pool_body,
        out_type=jax.ShapeDtypeStruct((_B, _D), jnp.float32),
        mesh=mesh,
        scratch_types=[
            pltpu.VMEM((_BPW, _S), jnp.int32),
            pltpu.VMEM((_S, _BPW), jnp.int32),
            pltpu.VMEM((_BPW, _D), jnp.float32),
            pltpu.SemaphoreType.DMA((_W,)),
        ],
        compiler_params=pltpu.CompilerParams(use_tc_tiling_on_sc=False,
                                             needs_layout_passes=False),
    )(x, table_lin)


def _fc_body(acc_ref, wfc_ref, bfc_ref, wfc1_ref, bfc1_ref, out_ref):
    ht = lax.dot_general(wfc_ref[...], acc_ref[...], (((1,), (1,)), ((), ())),
                         preferred_element_type=jnp.float32) + bfc_ref[...]
    ht = jnp.maximum(ht, 0.0)
    out_ref[...] = lax.dot_general(
        wfc1_ref[...], ht, (((1,), (0,)), ((), ())),
        preferred_element_type=jnp.float32) + bfc1_ref[...]


@jax.jit
def _fc(pooled, wfc, bfc_col, wfc1, bfc1_col):
    nc = wfc1.shape[0]
    return pl.pallas_call(
        _fc_body,
        out_shape=jax.ShapeDtypeStruct((nc, _B), jnp.float32),
    )(pooled, wfc, bfc_col, wfc1, bfc1_col)


def kernel(x, table, W_fc, b_fc, W_fc1, b_fc1):
    packed = _relayout_table(table.T)
    table_lin = packed.reshape(_VPAD, _D)
    pooled = _pool(x.astype(jnp.int32), table_lin)
    return _fc(pooled, W_fc, b_fc.reshape(-1, 1), W_fc1,
               b_fc1.reshape(-1, 1)).T


# speedup vs baseline: 1.0220x; 1.0220x over previous
import jax
import jax.numpy as jnp
from jax import lax
from jax.experimental import pallas as pl
from jax.experimental.pallas import tpu as pltpu
from jax.experimental.pallas import tpu_sc as plsc

_VOCAB = 1000000
_D = 64
_B = 4096
_S = 200

_TBLK = 8192
_NBLK = 62
_LA = _TBLK * _NBLK
_LB = _TBLK * (_NBLK - 1)
_VPAD = 2 * _LA

_NC = 2
_NS = 16
_NW = _NC * _NS
_BPW = _B // _NW
_W = 16


def _tr_body(ta_ref, tb_ref, out_ref):
    out_ref[...] = jnp.transpose(
        jnp.concatenate([ta_ref[...], tb_ref[...]], axis=0))


@jax.jit
def _relayout_table(table_t):
    return pl.pallas_call(
        _tr_body,
        grid=(_NBLK,),
        in_specs=[
            pl.BlockSpec((_D, _TBLK), lambda g: (0, g)),
            pl.BlockSpec((_D, _TBLK), lambda g: (0, g + _NBLK - 1)),
        ],
        out_specs=pl.BlockSpec((_TBLK, 2 * _D), lambda g: (g, 0)),
        out_shape=jax.ShapeDtypeStruct((_LA, 2 * _D), jnp.float32),
    )(table_t, table_t)


def _pool_body(x_hbm, table_hbm, out_hbm, x_v, xt_v, acc_v, sems):
    wid = lax.axis_index("s") * _NC + lax.axis_index("c")
    pltpu.sync_copy(x_hbm.at[pl.ds(wid * _BPW, _BPW)], x_v)

    zeros = jnp.zeros((16,), jnp.float32)

    def zero_row(b, c):
        for ch in range(_D // 16):
            acc_v[b, pl.ds(ch * 16, 16)] = zeros
        return c

    lax.fori_loop(0, _BPW, zero_row, 0)

    iota = lax.iota(jnp.int32, 16)

    def fire(s, slot):
        pltpu.async_copy(table_hbm.at[xt_v.at[s]], acc_v, sems.at[slot],
                         add=True)

    def ring_wait(slot):
        pltpu.make_async_copy(table_hbm.at[xt_v.at[0]], acc_v,
                              sems.at[slot]).wait()

    def step(s, c):
        col_idx = jnp.full((16,), 0, jnp.int32) + s
        for ch in range(_BPW // 16):
            row_idx = iota + (ch * 16)
            v = plsc.load_gather(x_v, [row_idx, col_idx])
            t = v + v
            vp = jnp.where(v < _LA, t, t - (2 * _LB - 1))
            xt_v[s, pl.ds(ch * 16, 16)] = vp
        slot = lax.rem(s, _W)

        @pl.when(s >= _W)
        def _():
            ring_wait(slot)

        fire(s, slot)
        return c

    lax.fori_loop(0, _S, step, 0)
    for j in range(_W):
        ring_wait(j)

    pltpu.sync_copy(acc_v, out_hbm.at[pl.ds(wid * _BPW, _BPW)])


@jax.jit
def _pool(x, table_lin):
    mesh = plsc.VectorSubcoreMesh(
        core_axis_name="c", subcore_axis_name="s", num_cores=_NC,
        num_subcores=_NS)
    return pl.kernel(
        _pool_body,
        out_type=jax.ShapeDtypeStruct((_B, _D), jnp.float32),
        mesh=mesh,
        scratch_types=[
            pltpu.VMEM((_BPW, _S), jnp.int32),
            pltpu.VMEM((_S, _BPW), jnp.int32),
            pltpu.VMEM((_BPW, _D), jnp.float32),
            pltpu.SemaphoreType.DMA((_W,)),
        ],
        compiler_params=pltpu.CompilerParams(use_tc_tiling_on_sc=False,
                                             needs_layout_passes=False),
    )(x, table_lin)


def _fc_body(acc_ref, wfc_ref, bfc_ref, wfc1_ref, bfc1_ref, out_ref):
    ht = lax.dot_general(wfc_ref[...], acc_ref[...], (((1,), (1,)), ((), ())),
                         preferred_element_type=jnp.float32) + bfc_ref[...]
    ht = jnp.maximum(ht, 0.0)
    out_ref[...] = lax.dot_general(
        wfc1_ref[...], ht, (((1,), (0,)), ((), ())),
        preferred_element_type=jnp.float32) + bfc1_ref[...]


@jax.jit
def _fc(pooled, wfc, bfc_col, wfc1, bfc1_col):
    nc = wfc1.shape[0]
    return pl.pallas_call(
        _fc_body,
        out_shape=jax.ShapeDtypeStruct((nc, _B), jnp.float32),
    )(pooled, wfc, bfc_col, wfc1, bfc1_col)


def kernel(x, table, W_fc, b_fc, W_fc1, b_fc1):
    packed = _relayout_table(table.T)
    table_lin = packed.reshape(_VPAD, _D)
    pooled = _pool(x.astype(jnp.int32), table_lin)
    return _fc(pooled, W_fc, b_fc.reshape(-1, 1), W_fc1,
               b_fc1.reshape(-1, 1)).T
